# Initial kernel scaffold; baseline (speedup 1.0000x reference)
import jax
import jax.numpy as jnp
from jax.experimental import pallas as pl


def kernel(sqrt_alphas_cumprod, t):
    raise NotImplementedError("write your pallas kernel here")


# speedup vs baseline: 4.5962x; 4.5962x over previous
import functools

import jax
import jax.numpy as jnp
from jax import lax
from jax.experimental import pallas as pl
from jax.experimental.pallas import tpu as pltpu
from jax.experimental.pallas import tpu_sc as plsc

_T = 1000
_TPAD = 1024
_BATCH = 16384
_NC = 2
_NS = 16
_NW = _NC * _NS
_BPW = _BATCH // _NW
_L = 16
_CHUNKS = _BPW // _L


def _make_gather():
    mesh = plsc.VectorSubcoreMesh(core_axis_name="c", subcore_axis_name="s")

    @functools.partial(
        pl.kernel,
        mesh=mesh,
        out_type=jax.ShapeDtypeStruct((_BATCH,), jnp.float32),
        scratch_types=[
            pltpu.VMEM((_TPAD,), jnp.float32),
            pltpu.VMEM((_BPW,), jnp.int32),
            pltpu.VMEM((_BPW,), jnp.float32),
        ],
        compiler_params=pltpu.CompilerParams(needs_layout_passes=False),
    )
    def gather_kernel(table_hbm, t_hbm, out_hbm, table_v, idx_v, res_v):
        wid = lax.axis_index("s") * _NC + lax.axis_index("c")
        base = wid * _BPW
        pltpu.sync_copy(table_hbm, table_v)
        pltpu.sync_copy(t_hbm.at[pl.ds(base, _BPW)], idx_v)
        for i in range(_CHUNKS):
            idx = idx_v[pl.ds(i * _L, _L)]
            res_v[pl.ds(i * _L, _L)] = plsc.load_gather(table_v, [idx])
        pltpu.sync_copy(res_v, out_hbm.at[pl.ds(base, _BPW)])

    return gather_kernel


_gather = _make_gather()


def kernel(sqrt_alphas_cumprod, t):
    table = jnp.pad(sqrt_alphas_cumprod, (0, _TPAD - _T))
    out = _gather(table, t)
    return out.reshape(-1, 1, 1)


# speedup vs baseline: 4.6569x; 1.0132x over previous
import functools

import jax
import jax.numpy as jnp
from jax import lax
from jax.experimental import pallas as pl
from jax.experimental.pallas import tpu as pltpu
from jax.experimental.pallas import tpu_sc as plsc

_T = 1000
_BATCH = 16384
_NC = 2
_NS = 16
_NW = _NC * _NS
_BPW = _BATCH // _NW
_L = 16
_CHUNKS = _BPW // _L


def _make_gather():
    mesh = plsc.VectorSubcoreMesh(core_axis_name="c", subcore_axis_name="s")

    @functools.partial(
        pl.kernel,
        mesh=mesh,
        out_type=jax.ShapeDtypeStruct((_BATCH,), jnp.float32),
        scratch_types=[
            pltpu.VMEM((_T,), jnp.float32),
            pltpu.VMEM((_BPW,), jnp.int32),
            pltpu.VMEM((_BPW,), jnp.float32),
            pltpu.SemaphoreType.DMA,
            pltpu.SemaphoreType.DMA,
        ],
        compiler_params=pltpu.CompilerParams(needs_layout_passes=False),
    )
    def gather_kernel(table_hbm, t_hbm, out_hbm, table_v, idx_v, res_v,
                      sem_a, sem_b):
        wid = lax.axis_index("s") * _NC + lax.axis_index("c")
        base = wid * _BPW
        cp_tab = pltpu.async_copy(table_hbm, table_v, sem_a)
        cp_idx = pltpu.async_copy(t_hbm.at[pl.ds(base, _BPW)], idx_v, sem_b)
        cp_tab.wait()
        cp_idx.wait()
        for i in range(_CHUNKS):
            idx = idx_v[pl.ds(i * _L, _L)]
            res_v[pl.ds(i * _L, _L)] = plsc.load_gather(table_v, [idx])
        pltpu.sync_copy(res_v, out_hbm.at[pl.ds(base, _BPW)])

    return gather_kernel


_gather = _make_gather()


def kernel(sqrt_alphas_cumprod, t):
    out = _gather(sqrt_alphas_cumprod, t)
    return out.reshape(-1, 1, 1)


# speedup vs baseline: 4.7657x; 1.0233x over previous
import functools

import jax
import jax.numpy as jnp
from jax import lax
from jax.experimental import pallas as pl
from jax.experimental.pallas import tpu as pltpu
from jax.experimental.pallas import tpu_sc as plsc

_T = 1000
_BATCH = 16384
_NC = 2
_NS = 16
_NW = _NC * _NS
_BPW = _BATCH // _NW
_L = 16
_CHUNKS = _BPW // _L


def _make_gather():
    mesh = plsc.VectorSubcoreMesh(core_axis_name="c", subcore_axis_name="s")

    @functools.partial(
        pl.kernel,
        mesh=mesh,
        out_type=jax.ShapeDtypeStruct((_BATCH,), jnp.float32),
        scratch_types=[
            pltpu.VMEM((_T,), jnp.float32),
            pltpu.VMEM((_BPW,), jnp.int32),
            pltpu.VMEM((_BPW,), jnp.float32),
            pltpu.SemaphoreType.DMA,
            pltpu.SemaphoreType.DMA,
        ],
        compiler_params=pltpu.CompilerParams(needs_layout_passes=False),
    )
    def gather_kernel(table_hbm, t_hbm, out_hbm, table_v, idx_v, res_v,
                      sem_a, sem_b):
        wid = lax.axis_index("s") * _NC + lax.axis_index("c")
        base = wid * _BPW
        cp_tab = pltpu.async_copy(table_hbm, table_v, sem_a)
        cp_idx = pltpu.async_copy(t_hbm.at[pl.ds(base, _BPW)], idx_v, sem_b)
        cp_tab.wait()
        cp_idx.wait()
        def body(i, carry):
            off = i * _L
            idx = idx_v[pl.ds(off, _L)]
            res_v[pl.ds(off, _L)] = plsc.load_gather(table_v, [idx])
            return carry

        lax.fori_loop(0, _CHUNKS, body, 0)
        pltpu.sync_copy(res_v, out_hbm.at[pl.ds(base, _BPW)])

    return gather_kernel


_gather = _make_gather()


def kernel(sqrt_alphas_cumprod, t):
    out = _gather(sqrt_alphas_cumprod, t)
    return out.reshape(-1, 1, 1)


# speedup vs baseline: 5.1264x; 1.0757x over previous
import functools

import jax
import jax.numpy as jnp
from jax import lax
from jax.experimental import pallas as pl
from jax.experimental.pallas import tpu as pltpu
from jax.experimental.pallas import tpu_sc as plsc

_T = 1000
_BATCH = 16384
_NC = 1
_NS = 16
_NW = _NC * _NS
_BPW = _BATCH // _NW
_L = 16
_CHUNKS = _BPW // _L


def _make_gather():
    mesh = plsc.VectorSubcoreMesh(core_axis_name="c", subcore_axis_name="s",
                                  num_cores=_NC)

    @functools.partial(
        pl.kernel,
        mesh=mesh,
        out_type=jax.ShapeDtypeStruct((_BATCH,), jnp.float32),
        scratch_types=[
            pltpu.VMEM((_T,), jnp.float32),
            pltpu.VMEM((_BPW,), jnp.int32),
            pltpu.VMEM((_BPW,), jnp.float32),
            pltpu.SemaphoreType.DMA,
            pltpu.SemaphoreType.DMA,
        ],
        compiler_params=pltpu.CompilerParams(needs_layout_passes=False),
    )
    def gather_kernel(table_hbm, t_hbm, out_hbm, table_v, idx_v, res_v,
                      sem_a, sem_b):
        wid = lax.axis_index("s") * _NC + lax.axis_index("c")
        base = wid * _BPW
        cp_tab = pltpu.async_copy(table_hbm, table_v, sem_a)
        cp_idx = pltpu.async_copy(t_hbm.at[pl.ds(base, _BPW)], idx_v, sem_b)
        cp_tab.wait()
        cp_idx.wait()
        def body(i, carry):
            off = i * _L
            idx = idx_v[pl.ds(off, _L)]
            res_v[pl.ds(off, _L)] = plsc.load_gather(table_v, [idx])
            return carry

        lax.fori_loop(0, _CHUNKS, body, 0)
        pltpu.sync_copy(res_v, out_hbm.at[pl.ds(base, _BPW)])

    return gather_kernel


_gather = _make_gather()


def kernel(sqrt_alphas_cumprod, t):
    out = _gather(sqrt_alphas_cumprod, t)
    return out.reshape(-1, 1, 1)
